# baseline (speedup 1.0000x reference)
import jax
import jax.numpy as jnp
from jax.experimental import pallas as pl

D_MODEL = 768
RANK = 64
N_EXPERT = 64
S_TILE = 1024
NG = 16
GROUPS = N_EXPERT // NG


def _main_body(x_ref, rwt_ref, wflat_ref, expand_ref, reduce_ref,
               out_ref, idx_ref, w_out_ref):
    x = x_ref[...]

    scores = jax.lax.dot_general(
        x, rwt_ref[...], (((1,), (0,)), ((), ())),
        preferred_element_type=jnp.float32)

    iota = jax.lax.broadcasted_iota(jnp.int32, (S_TILE, N_EXPERT), 1)
    m1 = jnp.max(scores, axis=1, keepdims=True)
    i1 = jnp.min(jnp.where(scores == m1, iota, N_EXPERT), axis=1,
                 keepdims=True)
    masked = jnp.where(iota == i1, -jnp.inf, scores)
    m2 = jnp.max(masked, axis=1, keepdims=True)
    i2 = jnp.min(jnp.where(masked == m2, iota, N_EXPERT), axis=1,
                 keepdims=True)

    e = jnp.exp(m2 - m1)
    denom = 1.0 + e
    w1 = 1.0 / denom
    w2 = e / denom

    idx_ref[...] = jnp.concatenate([i1, i2], axis=1)
    w_out_ref[...] = jnp.concatenate([w1, w2], axis=1)

    comb = jnp.where(iota == i1, w1, 0.0) + jnp.where(iota == i2, w2, 0.0)
    comb_bf = comb.astype(jnp.bfloat16)

    x_bf = x.astype(jnp.bfloat16)
    acc = jnp.zeros((S_TILE, RANK), dtype=jnp.float32)
    for g in range(GROUPS):
        sl = slice(NG * RANK * g, NG * RANK * (g + 1))
        combfull = jax.lax.dot_general(
            comb_bf, expand_ref[:, sl], (((1,), (0,)), ((), ())),
            preferred_element_type=jnp.float32).astype(jnp.bfloat16)
        proj = jax.lax.dot_general(
            x_bf, wflat_ref[:, sl], (((1,), (0,)), ((), ())),
            preferred_element_type=jnp.float32)
        cp = proj.astype(jnp.bfloat16) * combfull
        acc = acc + jax.lax.dot_general(
            cp, reduce_ref[sl, :], (((1,), (0,)), ((), ())),
            preferred_element_type=jnp.float32)
    out_ref[...] = acc


@jax.jit
def kernel(x, router_w, compress_neurons):
    b, s, d = x.shape
    xs = x.reshape(s, d)
    rwt = router_w.T

    cols = N_EXPERT * RANK
    w_bf = compress_neurons.astype(jnp.bfloat16)
    eye = jnp.eye(N_EXPERT, dtype=jnp.bfloat16)
    wflat = jnp.einsum('ndr,nm->dmr', w_bf, eye,
                       preferred_element_type=jnp.float32)
    wflat = wflat.astype(jnp.bfloat16).reshape(d, cols)

    c_iota = jnp.arange(cols, dtype=jnp.int32)
    expand = (jnp.arange(N_EXPERT, dtype=jnp.int32)[:, None]
              == (c_iota[None, :] // RANK)).astype(jnp.bfloat16)
    reduce = ((c_iota[:, None] % RANK)
              == jnp.arange(RANK, dtype=jnp.int32)[None, :]
              ).astype(jnp.bfloat16)

    grid = (s // S_TILE,)
    out, idx, w = pl.pallas_call(
        _main_body,
        grid=grid,
        in_specs=[
            pl.BlockSpec((S_TILE, d), lambda i: (i, 0)),
            pl.BlockSpec((d, N_EXPERT), lambda i: (0, 0)),
            pl.BlockSpec((d, cols), lambda i: (0, 0)),
            pl.BlockSpec((N_EXPERT, cols), lambda i: (0, 0)),
            pl.BlockSpec((cols, RANK), lambda i: (0, 0)),
        ],
        out_specs=[
            pl.BlockSpec((S_TILE, RANK), lambda i: (i, 0)),
            pl.BlockSpec((S_TILE, 2), lambda i: (i, 0)),
            pl.BlockSpec((S_TILE, 2), lambda i: (i, 0)),
        ],
        out_shape=[
            jax.ShapeDtypeStruct((s, RANK), jnp.float32),
            jax.ShapeDtypeStruct((s, 2), jnp.int32),
            jax.ShapeDtypeStruct((s, 2), jnp.float32),
        ],
    )(xs, rwt, wflat, expand, reduce)
    return (out.reshape(b, s, RANK), idx.reshape(b, s, 2),
            w.reshape(b, s, 2))


# speedup vs baseline: 1.4449x; 1.4449x over previous
import jax
import jax.numpy as jnp
from jax.experimental import pallas as pl

D_MODEL = 768
RANK = 64
N_EXPERT = 64
S_TILE = 1024
NG = 16
GROUPS = N_EXPERT // NG


def _main_body(x_ref, rwt_ref, wflat_ref, expand_ref, reduce_ref,
               out_ref, idx_ref, w_out_ref):
    x = x_ref[...]

    scores = jax.lax.dot_general(
        x, rwt_ref[...], (((1,), (0,)), ((), ())),
        preferred_element_type=jnp.float32)

    iota = jax.lax.broadcasted_iota(jnp.int32, (S_TILE, N_EXPERT), 1)
    m1 = jnp.max(scores, axis=1, keepdims=True)
    i1 = jnp.min(jnp.where(scores == m1, iota, N_EXPERT), axis=1,
                 keepdims=True)
    masked = jnp.where(iota == i1, -jnp.inf, scores)
    m2 = jnp.max(masked, axis=1, keepdims=True)
    i2 = jnp.min(jnp.where(masked == m2, iota, N_EXPERT), axis=1,
                 keepdims=True)

    e = jnp.exp(m2 - m1)
    denom = 1.0 + e
    w1 = 1.0 / denom
    w2 = e / denom

    idx_ref[...] = jnp.concatenate([i1, i2], axis=1)
    w_out_ref[...] = jnp.concatenate([w1, w2], axis=1)

    comb = jnp.where(iota == i1, w1, 0.0) + jnp.where(iota == i2, w2, 0.0)
    comb_bf = comb.astype(jnp.bfloat16)

    x_bf = x.astype(jnp.bfloat16)
    acc = jnp.zeros((S_TILE, RANK), dtype=jnp.float32)
    for g in range(GROUPS):
        sl = slice(NG * RANK * g, NG * RANK * (g + 1))
        combfull = jax.lax.dot_general(
            comb_bf, expand_ref[:, sl], (((1,), (0,)), ((), ())),
            preferred_element_type=jnp.float32).astype(jnp.bfloat16)
        proj = jax.lax.dot_general(
            x_bf, wflat_ref[:, sl], (((1,), (0,)), ((), ())),
            preferred_element_type=jnp.float32)
        cp = proj.astype(jnp.bfloat16) * combfull
        acc = acc + jax.lax.dot_general(
            cp, reduce_ref[sl, :], (((1,), (0,)), ((), ())),
            preferred_element_type=jnp.float32)
    out_ref[...] = acc


@jax.jit
def kernel(x, router_w, compress_neurons):
    b, s, d = x.shape
    xs = x.reshape(s, d)
    rwt = router_w.T

    cols = N_EXPERT * RANK
    wflat = compress_neurons.astype(jnp.bfloat16).transpose(1, 0, 2)
    wflat = wflat.reshape(d, cols)

    c_iota = jnp.arange(cols, dtype=jnp.int32)
    expand = (jnp.arange(N_EXPERT, dtype=jnp.int32)[:, None]
              == (c_iota[None, :] // RANK)).astype(jnp.bfloat16)
    reduce = ((c_iota[:, None] % RANK)
              == jnp.arange(RANK, dtype=jnp.int32)[None, :]
              ).astype(jnp.bfloat16)

    grid = (s // S_TILE,)
    out, idx, w = pl.pallas_call(
        _main_body,
        grid=grid,
        in_specs=[
            pl.BlockSpec((S_TILE, d), lambda i: (i, 0)),
            pl.BlockSpec((d, N_EXPERT), lambda i: (0, 0)),
            pl.BlockSpec((d, cols), lambda i: (0, 0)),
            pl.BlockSpec((N_EXPERT, cols), lambda i: (0, 0)),
            pl.BlockSpec((cols, RANK), lambda i: (0, 0)),
        ],
        out_specs=[
            pl.BlockSpec((S_TILE, RANK), lambda i: (i, 0)),
            pl.BlockSpec((S_TILE, 2), lambda i: (i, 0)),
            pl.BlockSpec((S_TILE, 2), lambda i: (i, 0)),
        ],
        out_shape=[
            jax.ShapeDtypeStruct((s, RANK), jnp.float32),
            jax.ShapeDtypeStruct((s, 2), jnp.int32),
            jax.ShapeDtypeStruct((s, 2), jnp.float32),
        ],
    )(xs, rwt, wflat, expand, reduce)
    return (out.reshape(b, s, RANK), idx.reshape(b, s, 2),
            w.reshape(b, s, 2))
